# Initial kernel scaffold; baseline (speedup 1.0000x reference)
import jax
import jax.numpy as jnp
from jax.experimental import pallas as pl


def kernel(x, edge_index, batch, W1_rel, b1, W1_root, W2_rel, b2, W2_root):
    raise NotImplementedError("write your pallas kernel here")


# speedup vs baseline: 18.3639x; 18.3639x over previous
import functools

import jax
import jax.numpy as jnp
from jax import lax
from jax.experimental import pallas as pl
from jax.experimental.pallas import tpu as pltpu
from jax.experimental.pallas import tpu_sc as plsc

N = 50000
E = 800000
D_IN = 64

NC = 2
NS = 16
NW = NC * NS
N_PAD = 50176
ROWS = N_PAD // NS
E_W = 25088
E_PAD = NW * E_W
BLK = 1024
GRID = N_PAD // BLK

_MESH = plsc.VectorSubcoreMesh(core_axis_name="c", subcore_axis_name="s")
_CP_SC = pltpu.CompilerParams(use_tc_tiling_on_sc=False)


def _sc_scatter2_body(t1, t3, srcr, dstr, zrows, o1, o2,
                      a1, a2, srcv, dstv, vals, stg, gsem):
    cid = lax.axis_index("c")
    sid = lax.axis_index("s")
    wid = cid * NS + sid
    sl = pl.ds(sid * ROWS, ROWS)
    pltpu.sync_copy(zrows, stg)
    pltpu.sync_copy(stg, a1.at[sl])
    pltpu.sync_copy(stg, a2.at[sl])
    pltpu.sync_copy(srcr.at[wid], srcv)
    pltpu.sync_copy(dstr.at[wid], dstv)
    plsc.subcore_barrier()
    pltpu.async_copy(t1.at[srcv], vals, gsem).wait()
    pltpu.sync_copy(vals, a1.at[dstv], add=True)
    pltpu.async_copy(t3.at[srcv], vals, gsem).wait()
    pltpu.sync_copy(vals, a2.at[dstv], add=True)
    plsc.subcore_barrier()
    pltpu.sync_copy(a1.at[sl], stg)
    pltpu.sync_copy(stg, o1.at[cid, sl])
    pltpu.sync_copy(a2.at[sl], stg)
    pltpu.sync_copy(stg, o2.at[cid, sl])


_sc_scatter2 = functools.partial(
    pl.kernel,
    out_type=[jax.ShapeDtypeStruct((NC, N_PAD), jnp.float32),
              jax.ShapeDtypeStruct((NC, N_PAD), jnp.float32)],
    mesh=_MESH,
    compiler_params=_CP_SC,
    scratch_types=[
        pltpu.VMEM_SHARED((N_PAD,), jnp.float32),
        pltpu.VMEM_SHARED((N_PAD,), jnp.float32),
        pltpu.VMEM((E_W,), jnp.int32),
        pltpu.VMEM((E_W,), jnp.int32),
        pltpu.VMEM((E_W,), jnp.float32),
        pltpu.VMEM((ROWS,), jnp.float32),
        pltpu.SemaphoreType.DMA,
    ],
)(_sc_scatter2_body)


def _sc_scatter1_body(t1, srcr, dstr, zrows, o1,
                      a1, srcv, dstv, vals, stg, gsem):
    cid = lax.axis_index("c")
    sid = lax.axis_index("s")
    wid = cid * NS + sid
    sl = pl.ds(sid * ROWS, ROWS)
    pltpu.sync_copy(zrows, stg)
    pltpu.sync_copy(stg, a1.at[sl])
    pltpu.sync_copy(srcr.at[wid], srcv)
    pltpu.sync_copy(dstr.at[wid], dstv)
    plsc.subcore_barrier()
    pltpu.async_copy(t1.at[srcv], vals, gsem).wait()
    pltpu.sync_copy(vals, a1.at[dstv], add=True)
    plsc.subcore_barrier()
    pltpu.sync_copy(a1.at[sl], stg)
    pltpu.sync_copy(stg, o1.at[cid, sl])


_sc_scatter1 = functools.partial(
    pl.kernel,
    out_type=jax.ShapeDtypeStruct((NC, N_PAD), jnp.float32),
    mesh=_MESH,
    compiler_params=_CP_SC,
    scratch_types=[
        pltpu.VMEM_SHARED((N_PAD,), jnp.float32),
        pltpu.VMEM((E_W,), jnp.int32),
        pltpu.VMEM((E_W,), jnp.int32),
        pltpu.VMEM((E_W,), jnp.float32),
        pltpu.VMEM((ROWS,), jnp.float32),
        pltpu.SemaphoreType.DMA,
    ],
)(_sc_scatter1_body)


def _im_i(i):
    return (0, i)


def _im_x(i):
    return (i, 0)


def _im_0(i):
    return (0, 0)


def _k1_body(x_ref, u13_ref, u24_ref, c_ref, q13_ref, q24_ref):
    a = x_ref[...]
    q13_ref[...] = jax.lax.dot_general(
        u13_ref[...], a, (((0,), (1,)), ((), ())),
        preferred_element_type=jnp.float32)
    q24_ref[...] = jax.lax.dot_general(
        u24_ref[...], a, (((0,), (1,)), ((), ())),
        preferred_element_type=jnp.float32) + c_ref[...]


def _k3_body(s1_ref, s3_ref, q_ref, o_ref):
    o_ref[0:1, :] = s1_ref[0:1, :] + s1_ref[1:2, :] + q_ref[0:1, :]
    o_ref[1:2, :] = s3_ref[0:1, :] + s3_ref[1:2, :] + q_ref[1:2, :]


def _k5_body(t_ref, p2_ref, o_ref):
    o_ref[...] = t_ref[0:1, :] + t_ref[1:2, :] + p2_ref[1:2, :]


def kernel(x, edge_index, batch, W1_rel, b1, W1_root, W2_rel, b2, W2_root):
    with jax.enable_x64(False):
        return _kernel_x32(x, edge_index, batch, W1_rel, b1, W1_root,
                           W2_rel, b2, W2_root)


def _kernel_x32(x, edge_index, batch, W1_rel, b1, W1_root, W2_rel, b2, W2_root):
    f32 = jnp.float32
    x = x.astype(f32)
    u1 = W1_rel.T @ W2_rel[0]
    u2 = W1_root.T @ W2_rel[0]
    u3 = W1_rel.T @ W2_root[0]
    u4 = W1_root.T @ W2_root[0]
    U13 = jnp.stack([u1, u3], axis=1).astype(f32)
    U24 = jnp.stack([u2, u4], axis=1).astype(f32)
    cvec = jnp.stack([jnp.dot(b1, W2_rel[0]),
                      jnp.dot(b1, W2_root[0]) + b2[0]]).astype(f32)[:, None]

    src = edge_index[0].astype(jnp.int32)
    dst = edge_index[1].astype(jnp.int32)
    pad = N + (jnp.arange(E_PAD - E, dtype=jnp.int32) % (N_PAD - N))
    srcr = jnp.concatenate([src, pad]).reshape(NW, E_W)
    dstr = jnp.concatenate([dst, pad]).reshape(NW, E_W)

    x_pad = jnp.pad(x, ((0, N_PAD - N), (0, 0)))
    zrows = jnp.zeros((ROWS,), f32)

    q13, q24c = pl.pallas_call(
        _k1_body,
        grid=(GRID,),
        in_specs=[
            pl.BlockSpec((BLK, D_IN), _im_x),
            pl.BlockSpec((D_IN, 2), _im_0),
            pl.BlockSpec((D_IN, 2), _im_0),
            pl.BlockSpec((2, 1), _im_0),
        ],
        out_specs=[pl.BlockSpec((2, BLK), _im_i),
                   pl.BlockSpec((2, BLK), _im_i)],
        out_shape=[jax.ShapeDtypeStruct((2, N_PAD), f32),
                   jax.ShapeDtypeStruct((2, N_PAD), f32)],
    )(x_pad, U13, U24, cvec)

    s1p, s3p = _sc_scatter2(q13[0], q13[1], srcr, dstr, zrows)

    p2 = pl.pallas_call(
        _k3_body,
        grid=(GRID,),
        in_specs=[pl.BlockSpec((2, BLK), _im_i)] * 3,
        out_specs=pl.BlockSpec((2, BLK), _im_i),
        out_shape=jax.ShapeDtypeStruct((2, N_PAD), f32),
    )(s1p, s3p, q24c)

    tp = _sc_scatter1(p2[0], srcr, dstr, zrows)

    out_row = pl.pallas_call(
        _k5_body,
        grid=(GRID,),
        in_specs=[pl.BlockSpec((2, BLK), _im_i)] * 2,
        out_specs=pl.BlockSpec((1, BLK), _im_i),
        out_shape=jax.ShapeDtypeStruct((1, N_PAD), f32),
    )(tp, p2)

    return out_row[0, :N, None]
